# baseline (speedup 1.0000x reference)
import functools

import jax
import jax.numpy as jnp
from jax import lax
from jax.experimental import pallas as pl
from jax.experimental.pallas import tpu as pltpu
from jax.experimental.pallas import tpu_sc as plsc

NC = 2
NS = 16
L = 16
NW = NC * NS


def kernel(users, movies, U, M, U_bias, M_bias):
    B = users.shape[0]
    F = U.shape[1]
    b_per_w = B // NW

    ub = U_bias.reshape(-1)
    mb = M_bias.reshape(-1)

    mesh = plsc.VectorSubcoreMesh(core_axis_name="c", subcore_axis_name="s")

    @functools.partial(
        pl.kernel,
        out_type=jax.ShapeDtypeStruct((B,), jnp.float32),
        mesh=mesh,
        compiler_params=pltpu.CompilerParams(needs_layout_passes=False,
                                             use_tc_tiling_on_sc=False),
        scratch_types=[
            pltpu.VMEM((b_per_w,), jnp.int32),
            pltpu.VMEM((b_per_w,), jnp.int32),
            pltpu.VMEM((b_per_w, F), jnp.float32),
            pltpu.VMEM((b_per_w, F), jnp.float32),
            pltpu.VMEM((b_per_w,), jnp.float32),
            pltpu.VMEM((b_per_w,), jnp.float32),
            pltpu.VMEM((b_per_w,), jnp.float32),
            pltpu.SemaphoreType.DMA,
        ],
    )
    def embed_dot_sc(users_h, movies_h, U_h, M_h, ub_h, mb_h, out_h,
                     uidx_v, midx_v, urows_v, mrows_v, ubias_v, mbias_v,
                     out_v, sem):
        wid = lax.axis_index("s") * NC + lax.axis_index("c")
        base = wid * b_per_w

        pltpu.sync_copy(users_h.at[pl.ds(base, b_per_w)], uidx_v)
        pltpu.sync_copy(movies_h.at[pl.ds(base, b_per_w)], midx_v)

        c1 = pltpu.async_copy(U_h.at[uidx_v], urows_v, sem)
        c2 = pltpu.async_copy(M_h.at[midx_v], mrows_v, sem)
        c3 = pltpu.async_copy(ub_h.at[uidx_v], ubias_v, sem)
        c4 = pltpu.async_copy(mb_h.at[midx_v], mbias_v, sem)
        c1.wait()
        c2.wait()
        c3.wait()
        c4.wait()

        lane = lax.iota(jnp.int32, L)
        n_groups = b_per_w // L

        def group_body(g, carry):
            rows = lane + g * L

            def f_body(f, acc):
                cols = jnp.full((L,), f, jnp.int32)
                uc = plsc.load_gather(urows_v, [rows, cols])
                mc = plsc.load_gather(mrows_v, [rows, cols])
                return acc + uc * mc

            acc = lax.fori_loop(0, F, f_body, jnp.zeros((L,), jnp.float32))
            x = (acc + plsc.load_gather(ubias_v, [rows])
                 + plsc.load_gather(mbias_v, [rows]))
            s = 1.0 / (1.0 + jnp.exp(-x))
            plsc.store_scatter(out_v, [rows], s * 6.0 - 0.5)
            return carry

        lax.fori_loop(0, n_groups, group_body, 0)
        pltpu.sync_copy(out_v, out_h.at[pl.ds(base, b_per_w)])

    return embed_dot_sc(users, movies, U, M, ub, mb)


# speedup vs baseline: 1.0357x; 1.0357x over previous
import functools

import jax
import jax.numpy as jnp
from jax import lax
from jax.experimental import pallas as pl
from jax.experimental.pallas import tpu as pltpu
from jax.experimental.pallas import tpu_sc as plsc

NC = 2
NS = 16
L = 16
NW = NC * NS
CHUNK = 128


def kernel(users, movies, U, M, U_bias, M_bias):
    B = users.shape[0]
    F = U.shape[1]
    b_per_w = B // NW
    n_chunks = b_per_w // CHUNK
    groups_per_chunk = CHUNK // L

    U2 = U.reshape(U.shape[0] // 2, 2 * F)
    M2 = M.reshape(M.shape[0] // 2, 2 * F)
    ub = U_bias.reshape(-1)
    mb = M_bias.reshape(-1)

    mesh = plsc.VectorSubcoreMesh(core_axis_name="c", subcore_axis_name="s")

    @functools.partial(
        pl.kernel,
        out_type=jax.ShapeDtypeStruct((B,), jnp.float32),
        mesh=mesh,
        compiler_params=pltpu.CompilerParams(needs_layout_passes=False),
        scratch_types=[
            pltpu.VMEM((b_per_w,), jnp.int32),
            pltpu.VMEM((b_per_w,), jnp.int32),
            pltpu.VMEM((n_chunks, CHUNK), jnp.int32),
            pltpu.VMEM((n_chunks, CHUNK), jnp.int32),
            pltpu.VMEM((b_per_w,), jnp.float32),
            pltpu.VMEM((b_per_w,), jnp.float32),
            pltpu.VMEM((CHUNK, 2 * F), jnp.float32),
            pltpu.VMEM((CHUNK, 2 * F), jnp.float32),
            pltpu.VMEM((CHUNK, 2 * F), jnp.float32),
            pltpu.VMEM((CHUNK, 2 * F), jnp.float32),
            pltpu.VMEM((b_per_w,), jnp.float32),
            pltpu.SemaphoreType.DMA,
            pltpu.SemaphoreType.DMA,
            pltpu.SemaphoreType.DMA,
        ],
    )
    def embed_dot_sc(users_h, movies_h, U2_h, M2_h, ub_h, mb_h, out_h,
                     uidx_v, midx_v, u2_v, m2_v, ubias_v, mbias_v,
                     ur0, ur1, mr0, mr1, out_v, semA, semB, semC):
        wid = lax.axis_index("s") * NC + lax.axis_index("c")
        base = wid * b_per_w

        pltpu.sync_copy(users_h.at[pl.ds(base, b_per_w)], uidx_v)
        pltpu.sync_copy(movies_h.at[pl.ds(base, b_per_w)], midx_v)

        cb1 = pltpu.async_copy(ub_h.at[uidx_v], ubias_v, semC)
        cb2 = pltpu.async_copy(mb_h.at[midx_v], mbias_v, semC)

        bufs = [(ur0, mr0, semA), (ur1, mr1, semB)]
        lane = lax.iota(jnp.int32, L)

        def do_shifts(c):
            def body(i, carry):
                s = pl.ds(c * CHUNK + i * L, L)
                u2_v[c, pl.ds(i * L, L)] = uidx_v[s] >> 1
                m2_v[c, pl.ds(i * L, L)] = midx_v[s] >> 1
                return carry
            lax.fori_loop(0, CHUNK // L, body, 0)

        def start(c):
            ub_, mb_, sem = bufs[c % 2]
            d1 = pltpu.async_copy(U2_h.at[u2_v.at[c]], ub_, sem)
            d2 = pltpu.async_copy(M2_h.at[m2_v.at[c]], mb_, sem)
            return d1, d2

        def compute(c):
            ubuf, mbuf, _ = bufs[c % 2]

            def group(g, carry):
                rows_l = lane + g * L
                rows_g = rows_l + c * CHUNK
                uoff = (plsc.load_gather(uidx_v, [rows_g]) & 1) << 6
                moff = (plsc.load_gather(midx_v, [rows_g]) & 1) << 6

                def f_body(f, acc):
                    k = (lane + f) & (F - 1)
                    uc = plsc.load_gather(ubuf, [rows_l, uoff + k])
                    mc = plsc.load_gather(mbuf, [rows_l, moff + k])
                    return acc + uc * mc

                acc = lax.fori_loop(0, F, f_body,
                                    jnp.zeros((L,), jnp.float32))
                x = (acc + plsc.load_gather(ubias_v, [rows_g])
                     + plsc.load_gather(mbias_v, [rows_g]))
                s = 1.0 / (1.0 + jnp.exp(-x))
                plsc.store_scatter(out_v, [rows_g], s * 6.0 - 0.5)
                return carry

            lax.fori_loop(0, groups_per_chunk, group, 0)

        do_shifts(0)
        pending = start(0)
        for c in range(n_chunks):
            if c + 1 < n_chunks:
                do_shifts(c + 1)
                nxt = start(c + 1)
            else:
                nxt = None
            pending[0].wait()
            pending[1].wait()
            if c == 0:
                cb1.wait()
                cb2.wait()
            compute(c)
            pending = nxt

        pltpu.sync_copy(out_v, out_h.at[pl.ds(base, b_per_w)])

    return embed_dot_sc(users, movies, U2, M2, ub, mb)


# speedup vs baseline: 1.5308x; 1.4779x over previous
import functools

import jax
import jax.numpy as jnp
from jax import lax
from jax.experimental import pallas as pl
from jax.experimental.pallas import tpu as pltpu
from jax.experimental.pallas import tpu_sc as plsc

NC = 2
NS = 16
L = 16
NW = NC * NS
CHUNK = 128


def kernel(users, movies, U, M, U_bias, M_bias):
    B = users.shape[0]
    F = U.shape[1]
    b_per_w = B // NW
    n_chunks = b_per_w // CHUNK
    groups_per_chunk = CHUNK // L

    ub = U_bias.reshape(-1)
    mb = M_bias.reshape(-1)

    mesh = plsc.VectorSubcoreMesh(core_axis_name="c", subcore_axis_name="s")

    @functools.partial(
        pl.kernel,
        out_type=jax.ShapeDtypeStruct((B,), jnp.float32),
        mesh=mesh,
        compiler_params=pltpu.CompilerParams(needs_layout_passes=False),
        scratch_types=[
            pltpu.VMEM((b_per_w,), jnp.int32),
            pltpu.VMEM((b_per_w,), jnp.int32),
            pltpu.VMEM((b_per_w,), jnp.float32),
            pltpu.VMEM((b_per_w,), jnp.float32),
            pltpu.VMEM((CHUNK, 64), jnp.float32),
            pltpu.VMEM((CHUNK, 64), jnp.float32),
            pltpu.VMEM((CHUNK, 64), jnp.float32),
            pltpu.VMEM((CHUNK, 64), jnp.float32),
            pltpu.VMEM((b_per_w,), jnp.float32),
            pltpu.SemaphoreType.DMA,
            pltpu.SemaphoreType.DMA,
            pltpu.SemaphoreType.DMA,
        ],
    )
    def embed_dot_sc(users_h, movies_h, U_h, M_h, ub_h, mb_h, out_h,
                     uidx_v, midx_v, ubias_v, mbias_v,
                     ur0, ur1, mr0, mr1, out_v, semA, semB, semC):
        wid = lax.axis_index("s") * NC + lax.axis_index("c")
        base = wid * b_per_w

        pltpu.sync_copy(users_h.at[pl.ds(base, b_per_w)], uidx_v)
        pltpu.sync_copy(movies_h.at[pl.ds(base, b_per_w)], midx_v)

        cb1 = pltpu.async_copy(ub_h.at[uidx_v], ubias_v, semC)
        cb2 = pltpu.async_copy(mb_h.at[midx_v], mbias_v, semC)

        bufs = [(ur0, mr0, semA), (ur1, mr1, semB)]
        lane = lax.iota(jnp.int32, L)

        def start(c):
            ubuf, mbuf, sem = bufs[c % 2]

            def body(g, carry):
                uvec = uidx_v[pl.ds(c * CHUNK + g * L, L)]
                mvec = midx_v[pl.ds(c * CHUNK + g * L, L)]
                for j in range(L):
                    r_u = uvec[j]
                    r_m = mvec[j]
                    i = g * L + j
                    pltpu.async_copy(U_h.at[pl.ds(r_u, 1)],
                                     ubuf.at[pl.ds(i, 1)], sem)
                    pltpu.async_copy(M_h.at[pl.ds(r_m, 1)],
                                     mbuf.at[pl.ds(i, 1)], sem)
                return carry

            lax.fori_loop(0, CHUNK // L, body, 0)

        def drain(c):
            ubuf, mbuf, sem = bufs[c % 2]
            pltpu.make_async_copy(U_h.at[pl.ds(0, CHUNK)], ubuf, sem).wait()
            pltpu.make_async_copy(U_h.at[pl.ds(0, CHUNK)], mbuf, sem).wait()

        def compute(c):
            ubuf, mbuf, _ = bufs[c % 2]

            def group(g, carry):
                rows_l = lane + g * L
                rows_g = rows_l + c * CHUNK

                def f_body(f, acc):
                    k = (lane + f) & (F - 1)
                    uc = plsc.load_gather(ubuf, [rows_l, k])
                    mc = plsc.load_gather(mbuf, [rows_l, k])
                    return acc + uc * mc

                acc = lax.fori_loop(0, F, f_body,
                                    jnp.zeros((L,), jnp.float32))
                x = (acc + plsc.load_gather(ubias_v, [rows_g])
                     + plsc.load_gather(mbias_v, [rows_g]))
                s = 1.0 / (1.0 + jnp.exp(-x))
                plsc.store_scatter(out_v, [rows_g], s * 6.0 - 0.5)
                return carry

            lax.fori_loop(0, groups_per_chunk, group, 0)

        start(0)
        for c in range(n_chunks):
            if c + 1 < n_chunks:
                start(c + 1)
            drain(c)
            if c == 0:
                cb1.wait()
                cb2.wait()
            compute(c)

        pltpu.sync_copy(out_v, out_h.at[pl.ds(base, b_per_w)])

    return embed_dot_sc(users, movies, U, M, ub, mb)


# speedup vs baseline: 1.7083x; 1.1160x over previous
import functools

import jax
import jax.numpy as jnp
from jax import lax
from jax.experimental import pallas as pl
from jax.experimental.pallas import tpu as pltpu
from jax.experimental.pallas import tpu_sc as plsc

NC = 2
NS = 16
L = 16
NW = NC * NS
CHUNK = 128


def kernel(users, movies, U, M, U_bias, M_bias):
    B = users.shape[0]
    F = U.shape[1]
    b_per_w = B // NW
    n_chunks = b_per_w // CHUNK
    groups_per_chunk = CHUNK // L

    ubT = U_bias.T
    mbT = M_bias.T

    mesh = plsc.VectorSubcoreMesh(core_axis_name="c", subcore_axis_name="s")

    @functools.partial(
        pl.kernel,
        out_type=jax.ShapeDtypeStruct((B,), jnp.float32),
        mesh=mesh,
        compiler_params=pltpu.CompilerParams(needs_layout_passes=False),
        scratch_types=[
            pltpu.VMEM((b_per_w,), jnp.int32),
            pltpu.VMEM((b_per_w,), jnp.int32),
            pltpu.VMEM((b_per_w,), jnp.float32),
            pltpu.VMEM((b_per_w,), jnp.float32),
            pltpu.VMEM((CHUNK, 64), jnp.float32),
            pltpu.VMEM((CHUNK, 64), jnp.float32),
            pltpu.VMEM((CHUNK, 64), jnp.float32),
            pltpu.VMEM((CHUNK, 64), jnp.float32),
            pltpu.VMEM((b_per_w,), jnp.float32),
            pltpu.SemaphoreType.DMA,
            pltpu.SemaphoreType.DMA,
            pltpu.SemaphoreType.DMA,
        ],
    )
    def embed_dot_sc(users_h, movies_h, U_h, M_h, ub_h, mb_h, out_h,
                     uidx_v, midx_v, ubias_v, mbias_v,
                     ur0, ur1, mr0, mr1, out_v, semA, semB, semC):
        wid = lax.axis_index("s") * NC + lax.axis_index("c")
        base = wid * b_per_w

        pltpu.sync_copy(users_h.at[pl.ds(base, b_per_w)], uidx_v)
        pltpu.sync_copy(movies_h.at[pl.ds(base, b_per_w)], midx_v)

        cb1 = pltpu.async_copy(ub_h.at[0].at[uidx_v], ubias_v, semC)
        cb2 = pltpu.async_copy(mb_h.at[0].at[midx_v], mbias_v, semC)

        bufs = [(ur0, mr0, semA), (ur1, mr1, semB)]
        lane = lax.iota(jnp.int32, L)

        def start(c):
            ubuf, mbuf, sem = bufs[c % 2]

            def body(g, carry):
                uvec = uidx_v[pl.ds(c * CHUNK + g * L, L)]
                mvec = midx_v[pl.ds(c * CHUNK + g * L, L)]
                for j in range(L):
                    r_u = uvec[j]
                    r_m = mvec[j]
                    i = g * L + j
                    pltpu.async_copy(U_h.at[pl.ds(r_u, 1)],
                                     ubuf.at[pl.ds(i, 1)], sem)
                    pltpu.async_copy(M_h.at[pl.ds(r_m, 1)],
                                     mbuf.at[pl.ds(i, 1)], sem)
                return carry

            lax.fori_loop(0, CHUNK // L, body, 0)

        def drain(c):
            ubuf, mbuf, sem = bufs[c % 2]
            pltpu.make_async_copy(U_h.at[pl.ds(0, CHUNK)], ubuf, sem).wait()
            pltpu.make_async_copy(U_h.at[pl.ds(0, CHUNK)], mbuf, sem).wait()

        def compute(c):
            ubuf, mbuf, _ = bufs[c % 2]

            def group(g, carry):
                rows_l = lane + g * L
                rows_g = rows_l + c * CHUNK

                def f_body(f, acc):
                    k = (lane + f) & (F - 1)
                    uc = plsc.load_gather(ubuf, [rows_l, k])
                    mc = plsc.load_gather(mbuf, [rows_l, k])
                    return acc + uc * mc

                acc = lax.fori_loop(0, F, f_body,
                                    jnp.zeros((L,), jnp.float32))
                x = (acc + plsc.load_gather(ubias_v, [rows_g])
                     + plsc.load_gather(mbias_v, [rows_g]))
                s = 1.0 / (1.0 + jnp.exp(-x))
                plsc.store_scatter(out_v, [rows_g], s * 6.0 - 0.5)
                return carry

            lax.fori_loop(0, groups_per_chunk, group, 0)

        start(0)
        for c in range(n_chunks):
            if c + 1 < n_chunks:
                start(c + 1)
            drain(c)
            if c == 0:
                cb1.wait()
                cb2.wait()
            compute(c)

        pltpu.sync_copy(out_v, out_h.at[pl.ds(base, b_per_w)])

    return embed_dot_sc(users, movies, U, M, ubT, mbT)


# speedup vs baseline: 1.7125x; 1.0024x over previous
import functools

import jax
import jax.numpy as jnp
from jax import lax
from jax.experimental import pallas as pl
from jax.experimental.pallas import tpu as pltpu
from jax.experimental.pallas import tpu_sc as plsc

NC = 2
NS = 16
L = 16
NW = NC * NS
CHUNK = 128


def kernel(users, movies, U, M, U_bias, M_bias):
    B = users.shape[0]
    F = U.shape[1]
    b_per_w = B // NW
    n_chunks = b_per_w // CHUNK
    groups_per_chunk = CHUNK // L

    U, M = lax.optimization_barrier((U, M))
    ubT = U_bias.T
    mbT = M_bias.T

    mesh = plsc.VectorSubcoreMesh(core_axis_name="c", subcore_axis_name="s")

    @functools.partial(
        pl.kernel,
        out_type=jax.ShapeDtypeStruct((B,), jnp.float32),
        mesh=mesh,
        compiler_params=pltpu.CompilerParams(needs_layout_passes=False),
        scratch_types=[
            pltpu.VMEM((b_per_w,), jnp.int32),
            pltpu.VMEM((b_per_w,), jnp.int32),
            pltpu.VMEM((b_per_w,), jnp.float32),
            pltpu.VMEM((b_per_w,), jnp.float32),
            pltpu.VMEM((CHUNK, 64), jnp.float32),
            pltpu.VMEM((CHUNK, 64), jnp.float32),
            pltpu.VMEM((CHUNK, 64), jnp.float32),
            pltpu.VMEM((CHUNK, 64), jnp.float32),
            pltpu.VMEM((b_per_w,), jnp.float32),
            pltpu.SemaphoreType.DMA,
            pltpu.SemaphoreType.DMA,
            pltpu.SemaphoreType.DMA,
        ],
    )
    def embed_dot_sc(users_h, movies_h, U_h, M_h, ub_h, mb_h, out_h,
                     uidx_v, midx_v, ubias_v, mbias_v,
                     ur0, ur1, mr0, mr1, out_v, semA, semB, semC):
        wid = lax.axis_index("s") * NC + lax.axis_index("c")
        base = wid * b_per_w

        pltpu.sync_copy(users_h.at[pl.ds(base, b_per_w)], uidx_v)
        pltpu.sync_copy(movies_h.at[pl.ds(base, b_per_w)], midx_v)

        cb1 = pltpu.async_copy(ub_h.at[0].at[uidx_v], ubias_v, semC)
        cb2 = pltpu.async_copy(mb_h.at[0].at[midx_v], mbias_v, semC)

        bufs = [(ur0, mr0, semA), (ur1, mr1, semB)]
        lane = lax.iota(jnp.int32, L)

        def start(c):
            ubuf, mbuf, sem = bufs[c % 2]

            def body(g, carry):
                uvec = uidx_v[pl.ds(c * CHUNK + g * L, L)]
                mvec = midx_v[pl.ds(c * CHUNK + g * L, L)]
                for j in range(L):
                    r_u = uvec[j]
                    r_m = mvec[j]
                    i = g * L + j
                    pltpu.async_copy(U_h.at[pl.ds(r_u, 1)],
                                     ubuf.at[pl.ds(i, 1)], sem)
                    pltpu.async_copy(M_h.at[pl.ds(r_m, 1)],
                                     mbuf.at[pl.ds(i, 1)], sem)
                return carry

            lax.fori_loop(0, CHUNK // L, body, 0)

        def drain(c):
            ubuf, mbuf, sem = bufs[c % 2]
            pltpu.make_async_copy(U_h.at[pl.ds(0, CHUNK)], ubuf, sem).wait()
            pltpu.make_async_copy(U_h.at[pl.ds(0, CHUNK)], mbuf, sem).wait()

        def compute(c):
            ubuf, mbuf, _ = bufs[c % 2]

            def group(g, carry):
                rows_l = lane + g * L
                rows_g = rows_l + c * CHUNK

                def f_body(f, acc):
                    k = (lane + f) & (F - 1)
                    uc = plsc.load_gather(ubuf, [rows_l, k])
                    mc = plsc.load_gather(mbuf, [rows_l, k])
                    return acc + uc * mc

                acc = lax.fori_loop(0, F, f_body,
                                    jnp.zeros((L,), jnp.float32))
                x = (acc + plsc.load_gather(ubias_v, [rows_g])
                     + plsc.load_gather(mbias_v, [rows_g]))
                s = 1.0 / (1.0 + jnp.exp(-x))
                plsc.store_scatter(out_v, [rows_g], s * 6.0 - 0.5)
                return carry

            lax.fori_loop(0, groups_per_chunk, group, 0)

        start(0)
        for c in range(n_chunks):
            if c + 1 < n_chunks:
                start(c + 1)
            drain(c)
            if c == 0:
                cb1.wait()
                cb2.wait()
            compute(c)

        pltpu.sync_copy(out_v, out_h.at[pl.ds(base, b_per_w)])

    return embed_dot_sc(users, movies, U, M, ubT, mbT)
